# baseline (speedup 1.0000x reference)
import jax
import jax.numpy as jnp
from jax.experimental import pallas as pl
from jax.experimental.pallas import tpu as pltpu


def _copy_body(hs_ref, pe_ref, m_ref, hs_out, pe_out, m_out, sem):
    c1 = pltpu.make_async_copy(hs_ref, hs_out, sem)
    c2 = pltpu.make_async_copy(pe_ref, pe_out, sem)
    c3 = pltpu.make_async_copy(m_ref, m_out, sem)
    c1.start()
    c2.start()
    c3.start()
    c1.wait()
    c2.wait()
    c3.wait()


def kernel(hidden_states, position_embeddings, attention_mask):
    b, q, h = hidden_states.shape
    hs2 = hidden_states.reshape(b, h)
    pe2 = position_embeddings.reshape(b, h)
    m2 = attention_mask.reshape(1, b)

    any_spec = pl.BlockSpec(memory_space=pl.MemorySpace.ANY)
    hs_o, pe_o, m_o = pl.pallas_call(
        _copy_body,
        in_specs=[any_spec, any_spec, any_spec],
        out_specs=[any_spec, any_spec, any_spec],
        out_shape=(
            jax.ShapeDtypeStruct(hs2.shape, hs2.dtype),
            jax.ShapeDtypeStruct(pe2.shape, pe2.dtype),
            jax.ShapeDtypeStruct(m2.shape, m2.dtype),
        ),
        scratch_shapes=[pltpu.SemaphoreType.DMA],
    )(hs2, pe2, m2)

    return (
        hs_o.reshape(hidden_states.shape),
        pe_o.reshape(position_embeddings.shape),
        m_o.reshape(attention_mask.shape),
    )


# speedup vs baseline: 5.9965x; 5.9965x over previous
import jax
import jax.numpy as jnp
from jax.experimental import pallas as pl
from jax.experimental.pallas import tpu as pltpu

_GRID = 8


def _copy_body(hs_ref, pe_ref, m_ref, hs_out, pe_out, m_out):
    hs_out[...] = hs_ref[...]
    pe_out[...] = pe_ref[...]
    m_out[...] = m_ref[...]


def kernel(hidden_states, position_embeddings, attention_mask):
    b, q, h = hidden_states.shape
    hs2 = hidden_states.reshape(b, h)
    pe2 = position_embeddings.reshape(b, h)
    m2 = attention_mask.reshape(1, b)

    rows = b // _GRID
    big_spec = pl.BlockSpec((rows, h), lambda i: (i, 0))
    m_spec = pl.BlockSpec((1, b), lambda i: (0, 0))

    hs_o, pe_o, m_o = pl.pallas_call(
        _copy_body,
        grid=(_GRID,),
        in_specs=[big_spec, big_spec, m_spec],
        out_specs=[big_spec, big_spec, m_spec],
        out_shape=(
            jax.ShapeDtypeStruct(hs2.shape, hs2.dtype),
            jax.ShapeDtypeStruct(pe2.shape, pe2.dtype),
            jax.ShapeDtypeStruct(m2.shape, m2.dtype),
        ),
    )(hs2, pe2, m2)

    return (
        hs_o.reshape(hidden_states.shape),
        pe_o.reshape(position_embeddings.shape),
        m_o.reshape(attention_mask.shape),
    )


# speedup vs baseline: 6.9196x; 1.1539x over previous
import jax
import jax.numpy as jnp
from jax.experimental import pallas as pl
from jax.experimental.pallas import tpu as pltpu

_CHUNKS = 8


def _copy_body(hs_hbm, pe_hbm, m_hbm, hs_out, pe_out, m_out,
               hs_v, pe_v, m_v, in_sems, out_sems, m_in_sem, m_out_sem):
    b = hs_hbm.shape[0]
    rows = b // _CHUNKS

    in_copies = []
    for i in range(_CHUNKS):
        sl = pl.ds(i * rows, rows)
        c_hs = pltpu.make_async_copy(hs_hbm.at[sl], hs_v.at[sl], in_sems.at[2 * i])
        c_pe = pltpu.make_async_copy(pe_hbm.at[sl], pe_v.at[sl], in_sems.at[2 * i + 1])
        c_hs.start()
        c_pe.start()
        in_copies.append((sl, c_hs, c_pe))
    c_m = pltpu.make_async_copy(m_hbm, m_v, m_in_sem)
    c_m.start()

    out_copies = []
    for i, (sl, c_hs, c_pe) in enumerate(in_copies):
        c_hs.wait()
        o_hs = pltpu.make_async_copy(hs_v.at[sl], hs_out.at[sl], out_sems.at[2 * i])
        o_hs.start()
        c_pe.wait()
        o_pe = pltpu.make_async_copy(pe_v.at[sl], pe_out.at[sl], out_sems.at[2 * i + 1])
        o_pe.start()
        out_copies.append(o_hs)
        out_copies.append(o_pe)
    c_m.wait()
    o_m = pltpu.make_async_copy(m_v, m_out, m_out_sem)
    o_m.start()

    for o in out_copies:
        o.wait()
    o_m.wait()


def kernel(hidden_states, position_embeddings, attention_mask):
    b, q, h = hidden_states.shape
    hs2 = hidden_states.reshape(b, h)
    pe2 = position_embeddings.reshape(b, h)
    m2 = attention_mask.reshape(1, b)

    any_spec = pl.BlockSpec(memory_space=pl.MemorySpace.ANY)
    hs_o, pe_o, m_o = pl.pallas_call(
        _copy_body,
        in_specs=[any_spec, any_spec, any_spec],
        out_specs=[any_spec, any_spec, any_spec],
        out_shape=(
            jax.ShapeDtypeStruct(hs2.shape, hs2.dtype),
            jax.ShapeDtypeStruct(pe2.shape, pe2.dtype),
            jax.ShapeDtypeStruct(m2.shape, m2.dtype),
        ),
        scratch_shapes=[
            pltpu.VMEM((b, h), jnp.float32),
            pltpu.VMEM((b, h), jnp.float32),
            pltpu.VMEM((1, b), jnp.float32),
            pltpu.SemaphoreType.DMA((2 * _CHUNKS,)),
            pltpu.SemaphoreType.DMA((2 * _CHUNKS,)),
            pltpu.SemaphoreType.DMA,
            pltpu.SemaphoreType.DMA,
        ],
    )(hs2, pe2, m2)

    return (
        hs_o.reshape(hidden_states.shape),
        pe_o.reshape(position_embeddings.shape),
        m_o.reshape(attention_mask.shape),
    )
